# Initial kernel scaffold; baseline (speedup 1.0000x reference)
import jax
import jax.numpy as jnp
from jax.experimental import pallas as pl


def kernel(node_species, embed_table):
    raise NotImplementedError("write your pallas kernel here")


# speedup vs baseline: 1.2262x; 1.2262x over previous
import functools

import jax
import jax.numpy as jnp
from jax import lax
from jax.experimental import pallas as pl
from jax.experimental.pallas import tpu as pltpu
from jax.experimental.pallas import tpu_sc as plsc

NUM_SPECIES = 119
EMBED_DIM = 128
N_NODES = 100000

NC, NS = 2, 16
NW = NC * NS
CHUNK = 448
NCHUNKS = 7
B_PER_W = CHUNK * NCHUNKS
B_PAD = B_PER_W * NW


def _embed_body(table_hbm, idx_hbm, out_hbm, idx_v, buf0, buf1, gs0, gs1,
                os0, os1):
    wid = lax.axis_index("s") * NC + lax.axis_index("c")
    base = wid * B_PER_W

    pltpu.sync_copy(idx_hbm.at[pl.ds(base, B_PER_W)], idx_v)

    bufs = (buf0, buf1)
    gsems = (gs0, gs1)
    osems = (os0, os1)

    def start_gather(c):
        return pltpu.async_copy(
            table_hbm.at[idx_v.at[pl.ds(c * CHUNK, CHUNK)]],
            bufs[c % 2],
            gsems[c % 2],
        )

    def start_out(c):
        return pltpu.async_copy(
            bufs[c % 2],
            out_hbm.at[pl.ds(base + c * CHUNK, CHUNK)],
            osems[c % 2],
        )

    gather = [None, None]
    out = [None, None]
    gather[0] = start_gather(0)
    for c in range(NCHUNKS):
        nxt = c + 1
        if nxt < NCHUNKS:
            if out[nxt % 2] is not None:
                out[nxt % 2].wait()
            gather[nxt % 2] = start_gather(nxt)
        gather[c % 2].wait()
        out[c % 2] = start_out(c)
    out[(NCHUNKS - 1) % 2].wait()
    if out[NCHUNKS % 2] is not None:
        out[NCHUNKS % 2].wait()


@jax.jit
def kernel(node_species, embed_table):
    idx = jnp.pad(node_species.astype(jnp.int32), (0, B_PAD - N_NODES))
    mesh = plsc.VectorSubcoreMesh(core_axis_name="c", subcore_axis_name="s")
    out = pl.kernel(
        _embed_body,
        out_type=jax.ShapeDtypeStruct((B_PAD, EMBED_DIM), jnp.float32),
        mesh=mesh,
        scratch_types=[
            pltpu.VMEM((B_PER_W,), jnp.int32),
            pltpu.VMEM((CHUNK, EMBED_DIM), jnp.float32),
            pltpu.VMEM((CHUNK, EMBED_DIM), jnp.float32),
            pltpu.SemaphoreType.DMA,
            pltpu.SemaphoreType.DMA,
            pltpu.SemaphoreType.DMA,
            pltpu.SemaphoreType.DMA,
        ],
    )(embed_table, idx)
    return out[:N_NODES]


# speedup vs baseline: 1.5807x; 1.2891x over previous
import jax
import jax.numpy as jnp
from jax import lax
from jax.experimental import pallas as pl
from jax.experimental.pallas import tpu as pltpu
from jax.experimental.pallas import tpu_sc as plsc

NUM_SPECIES = 119
EMBED_DIM = 128
N_NODES = 100000

NC, NS = 2, 16
NW = NC * NS
SPAN = 3128
CHUNK = 448
NFULL = 6
TAIL_A = SPAN - NFULL * CHUNK
TAIL_B = N_NODES - (NW - 1) * SPAN - NFULL * CHUNK


def _embed_body(table_hbm, idx_hbm, out_hbm, idx_v, buf0, buf1, gs0, gs1,
                os0, os1):
    wid = lax.axis_index("s") * NC + lax.axis_index("c")
    base = wid * SPAN
    is_last = wid == NW - 1

    pltpu.sync_copy(idx_hbm.at[pl.ds(base, NFULL * CHUNK)],
                    idx_v.at[pl.ds(0, NFULL * CHUNK)])
    tb = base + NFULL * CHUNK

    def stage_tail_a():
        pltpu.sync_copy(idx_hbm.at[pl.ds(tb, TAIL_A)],
                        idx_v.at[pl.ds(NFULL * CHUNK, TAIL_A)])

    def stage_tail_b():
        pltpu.sync_copy(idx_hbm.at[pl.ds(tb, TAIL_B)],
                        idx_v.at[pl.ds(NFULL * CHUNK, TAIL_B)])

    pl.when(jnp.logical_not(is_last))(stage_tail_a)
    pl.when(is_last)(stage_tail_b)

    bufs = (buf0, buf1)
    gsems = (gs0, gs1)
    osems = (os0, os1)

    def start_gather(c, n=CHUNK):
        return pltpu.async_copy(
            table_hbm.at[idx_v.at[pl.ds(c * CHUNK, n)]],
            bufs[c % 2].at[pl.ds(0, n)],
            gsems[c % 2],
        )

    def start_out(c, n=CHUNK):
        return pltpu.async_copy(
            bufs[c % 2].at[pl.ds(0, n)],
            out_hbm.at[pl.ds(base + c * CHUNK, n)],
            osems[c % 2],
        )

    gather = [None, None]
    out = [None, None]
    gather[0] = start_gather(0)
    for c in range(NFULL):
        nxt = c + 1
        if nxt < NFULL:
            if out[nxt % 2] is not None:
                out[nxt % 2].wait()
            gather[nxt % 2] = start_gather(nxt)
        gather[c % 2].wait()
        out[c % 2] = start_out(c)

    out[(NFULL - 2) % 2].wait()

    def tail_a():
        start_gather(NFULL, TAIL_A).wait()
        start_out(NFULL, TAIL_A).wait()

    def tail_b():
        start_gather(NFULL, TAIL_B).wait()
        start_out(NFULL, TAIL_B).wait()

    pl.when(jnp.logical_not(is_last))(tail_a)
    pl.when(is_last)(tail_b)

    out[(NFULL - 1) % 2].wait()


@jax.jit
def kernel(node_species, embed_table):
    idx = node_species.astype(jnp.int32)
    mesh = plsc.VectorSubcoreMesh(core_axis_name="c", subcore_axis_name="s")
    return pl.kernel(
        _embed_body,
        out_type=jax.ShapeDtypeStruct((N_NODES, EMBED_DIM), jnp.float32),
        mesh=mesh,
        scratch_types=[
            pltpu.VMEM((SPAN,), jnp.int32),
            pltpu.VMEM((CHUNK, EMBED_DIM), jnp.float32),
            pltpu.VMEM((CHUNK, EMBED_DIM), jnp.float32),
            pltpu.SemaphoreType.DMA,
            pltpu.SemaphoreType.DMA,
            pltpu.SemaphoreType.DMA,
            pltpu.SemaphoreType.DMA,
        ],
    )(embed_table, idx)


# speedup vs baseline: 5.3775x; 3.4021x over previous
import jax
import jax.numpy as jnp
from jax import lax
from jax.experimental import pallas as pl
from jax.experimental.pallas import tpu as pltpu
from jax.experimental.pallas import tpu_sc as plsc

NUM_SPECIES = 119
EMBED_DIM = 128
N_NODES = 100000

NC, NS = 2, 16
NW = NC * NS
SPAN = 3128
CHUNK = 448
NFULL = 6
TAIL_A = SPAN - NFULL * CHUNK
TAIL_B = N_NODES - (NW - 1) * SPAN - NFULL * CHUNK


def _embed_body(table_hbm, idx_hbm, out_hbm, table_sh, idx_v, buf0, buf1,
                gs0, gs1, os0, os1):
    wid = lax.axis_index("s") * NC + lax.axis_index("c")
    base = wid * SPAN
    is_last = wid == NW - 1

    def stage_table():
        pltpu.sync_copy(table_hbm, table_sh)

    pl.when(lax.axis_index("s") == 0)(stage_table)

    pltpu.sync_copy(idx_hbm.at[pl.ds(base, NFULL * CHUNK)],
                    idx_v.at[pl.ds(0, NFULL * CHUNK)])
    tb = base + NFULL * CHUNK

    def stage_tail_a():
        pltpu.sync_copy(idx_hbm.at[pl.ds(tb, TAIL_A)],
                        idx_v.at[pl.ds(NFULL * CHUNK, TAIL_A)])

    def stage_tail_b():
        pltpu.sync_copy(idx_hbm.at[pl.ds(tb, TAIL_B)],
                        idx_v.at[pl.ds(NFULL * CHUNK, TAIL_B)])

    pl.when(jnp.logical_not(is_last))(stage_tail_a)
    pl.when(is_last)(stage_tail_b)

    plsc.subcore_barrier()

    bufs = (buf0, buf1)
    gsems = (gs0, gs1)
    osems = (os0, os1)

    def start_gather(c, n=CHUNK):
        return pltpu.async_copy(
            table_sh.at[idx_v.at[pl.ds(c * CHUNK, n)]],
            bufs[c % 2].at[pl.ds(0, n)],
            gsems[c % 2],
        )

    def start_out(c, n=CHUNK):
        return pltpu.async_copy(
            bufs[c % 2].at[pl.ds(0, n)],
            out_hbm.at[pl.ds(base + c * CHUNK, n)],
            osems[c % 2],
        )

    gather = [None, None]
    out = [None, None]
    gather[0] = start_gather(0)
    for c in range(NFULL):
        nxt = c + 1
        if nxt < NFULL:
            if out[nxt % 2] is not None:
                out[nxt % 2].wait()
            gather[nxt % 2] = start_gather(nxt)
        gather[c % 2].wait()
        out[c % 2] = start_out(c)

    out[(NFULL - 2) % 2].wait()

    def tail_a():
        start_gather(NFULL, TAIL_A).wait()
        start_out(NFULL, TAIL_A).wait()

    def tail_b():
        start_gather(NFULL, TAIL_B).wait()
        start_out(NFULL, TAIL_B).wait()

    pl.when(jnp.logical_not(is_last))(tail_a)
    pl.when(is_last)(tail_b)

    out[(NFULL - 1) % 2].wait()


@jax.jit
def kernel(node_species, embed_table):
    idx = node_species.astype(jnp.int32)
    mesh = plsc.VectorSubcoreMesh(core_axis_name="c", subcore_axis_name="s")
    return pl.kernel(
        _embed_body,
        out_type=jax.ShapeDtypeStruct((N_NODES, EMBED_DIM), jnp.float32),
        mesh=mesh,
        scratch_types=[
            pltpu.VMEM_SHARED((NUM_SPECIES, EMBED_DIM), jnp.float32),
            pltpu.VMEM((SPAN,), jnp.int32),
            pltpu.VMEM((CHUNK, EMBED_DIM), jnp.float32),
            pltpu.VMEM((CHUNK, EMBED_DIM), jnp.float32),
            pltpu.SemaphoreType.DMA,
            pltpu.SemaphoreType.DMA,
            pltpu.SemaphoreType.DMA,
            pltpu.SemaphoreType.DMA,
        ],
    )(embed_table, idx)


# speedup vs baseline: 5.6072x; 1.0427x over previous
import jax
import jax.numpy as jnp
from jax import lax
from jax.experimental import pallas as pl
from jax.experimental.pallas import tpu as pltpu
from jax.experimental.pallas import tpu_sc as plsc

NUM_SPECIES = 119
EMBED_DIM = 128
N_NODES = 100000

NC, NS = 2, 16
NW = NC * NS
SPAN = 3128
CHUNK = 320
NBUF = 3
NFULL = SPAN // CHUNK
TAIL_A = SPAN - NFULL * CHUNK
TAIL_B = (N_NODES - (NW - 1) * SPAN) - NFULL * CHUNK


def _embed_body(table_hbm, idx_hbm, out_hbm, table_sh, idx_v, bufs_ref,
                *sems):
    gsems = sems[:NBUF]
    osems = sems[NBUF:]
    wid = lax.axis_index("s") * NC + lax.axis_index("c")
    base = wid * SPAN
    is_last = wid == NW - 1

    def stage_table():
        pltpu.sync_copy(table_hbm, table_sh)

    pl.when(lax.axis_index("s") == 0)(stage_table)

    pltpu.sync_copy(idx_hbm.at[pl.ds(base, NFULL * CHUNK)],
                    idx_v.at[pl.ds(0, NFULL * CHUNK)])
    tb = base + NFULL * CHUNK

    def stage_tail_a():
        pltpu.sync_copy(idx_hbm.at[pl.ds(tb, TAIL_A)],
                        idx_v.at[pl.ds(NFULL * CHUNK, TAIL_A)])

    def stage_tail_b():
        pltpu.sync_copy(idx_hbm.at[pl.ds(tb, TAIL_B)],
                        idx_v.at[pl.ds(NFULL * CHUNK, TAIL_B)])

    pl.when(jnp.logical_not(is_last))(stage_tail_a)
    pl.when(is_last)(stage_tail_b)

    plsc.subcore_barrier()

    def start_gather(c, n=CHUNK):
        return pltpu.async_copy(
            table_sh.at[idx_v.at[pl.ds(c * CHUNK, n)]],
            bufs_ref.at[c % NBUF].at[pl.ds(0, n)],
            gsems[c % NBUF],
        )

    def start_out(c, n=CHUNK):
        return pltpu.async_copy(
            bufs_ref.at[c % NBUF].at[pl.ds(0, n)],
            out_hbm.at[pl.ds(base + c * CHUNK, n)],
            osems[c % NBUF],
        )

    gather = [None] * NBUF
    out = [None] * NBUF
    for c in range(min(NBUF, NFULL)):
        gather[c % NBUF] = start_gather(c)
    for c in range(NFULL):
        nxt = c + NBUF
        gather[c % NBUF].wait()
        out[c % NBUF] = start_out(c)
        if nxt < NFULL:
            out[nxt % NBUF].wait()
            out[nxt % NBUF] = None
            gather[nxt % NBUF] = start_gather(nxt)

    tslot = NFULL % NBUF
    if out[tslot] is not None:
        out[tslot].wait()
        out[tslot] = None

    def tail_a():
        start_gather(NFULL, TAIL_A).wait()
        start_out(NFULL, TAIL_A).wait()

    def tail_b():
        start_gather(NFULL, TAIL_B).wait()
        start_out(NFULL, TAIL_B).wait()

    pl.when(jnp.logical_not(is_last))(tail_a)
    pl.when(is_last)(tail_b)

    for o in out:
        if o is not None:
            o.wait()


@jax.jit
def kernel(node_species, embed_table):
    idx = node_species.astype(jnp.int32)
    mesh = plsc.VectorSubcoreMesh(core_axis_name="c", subcore_axis_name="s")
    return pl.kernel(
        _embed_body,
        out_type=jax.ShapeDtypeStruct((N_NODES, EMBED_DIM), jnp.float32),
        mesh=mesh,
        scratch_types=[
            pltpu.VMEM_SHARED((NUM_SPECIES, EMBED_DIM), jnp.float32),
            pltpu.VMEM((SPAN,), jnp.int32),
            pltpu.VMEM((NBUF, CHUNK, EMBED_DIM), jnp.float32),
        ] + [pltpu.SemaphoreType.DMA] * (2 * NBUF),
    )(embed_table, idx)
